# baseline (speedup 1.0000x reference)
import functools

import jax
import jax.numpy as jnp
from jax import lax
from jax.experimental import pallas as pl
from jax.experimental.pallas import tpu as pltpu
from jax.experimental.pallas import tpu_sc as plsc

_HI = jax.lax.Precision.HIGHEST


def _topk_body(x1_ref, x2t_ref, idx_ref, wt_ref, *, S):
    b = pl.program_id(0)
    x1 = x1_ref[0]
    x2t = x2t_ref[0]
    d0 = x1[:, 0:1] - x2t[0:1, :]
    d1 = x1[:, 1:2] - x2t[1:2, :]
    d2c = x1[:, 2:3] - x2t[2:3, :]
    d = d0 * d0 + d1 * d1 + d2c * d2c

    inf = jnp.float32(jnp.inf)
    iota = lax.broadcasted_iota(jnp.int32, d.shape, 1)
    m1 = jnp.min(d, axis=1, keepdims=True)
    i1 = jnp.min(jnp.where(d <= m1, iota, S), axis=1, keepdims=True)
    dm2 = jnp.where(d <= m1, inf, d)
    m2 = jnp.min(dm2, axis=1, keepdims=True)
    i2 = jnp.min(jnp.where(dm2 <= m2, iota, S), axis=1, keepdims=True)
    dm3 = jnp.where(d <= m2, inf, d)
    m3 = jnp.min(dm3, axis=1, keepdims=True)
    i3 = jnp.min(jnp.where(dm3 <= m3, iota, S), axis=1, keepdims=True)

    r1 = 1.0 / (m1 + 1e-8)
    r2 = 1.0 / (m2 + 1e-8)
    r3 = 1.0 / (m3 + 1e-8)
    norm = r1 + r2 + r3
    base = b * S
    idx_ref[0] = jnp.concatenate([i1 + base, i2 + base, i3 + base], axis=1)
    wt_ref[0] = jnp.concatenate([r1, r2, r3], axis=1) / norm


def _p2w_body(p2_ref, w1bt_ref, out_ref):
    out_ref[0] = jnp.dot(p2_ref[0], w1bt_ref[...],
                         preferred_element_type=jnp.float32, precision=_HI)


def _sc_gather(table_hbm, idx_hbm, wts_hbm, out_hbm,
               idxall_v, wtsall_v, rows_v, obuf_v, gs0, gs1, ws0, ws1, *,
               pts_per_w, cp, f1, nc):
    wid = lax.axis_index("s") * nc + lax.axis_index("c")
    wbase = wid * pts_per_w
    nchunks = pts_per_w // cp
    gsems = (gs0, gs1)
    wsems = (ws0, ws1)

    pltpu.sync_copy(idx_hbm.at[pl.ds(3 * wbase, 3 * pts_per_w)], idxall_v)
    pltpu.sync_copy(wts_hbm.at[pl.ds(3 * wbase, 3 * pts_per_w)],
                    wtsall_v.at[pl.ds(0, 3 * pts_per_w)])

    def fire(c, buf):
        pltpu.make_async_copy(
            table_hbm.at[idxall_v.at[pl.ds(3 * cp * c, 3 * cp)]],
            rows_v.at[buf], gsems[buf]).start()

    def gather_wait(buf):
        pltpu.make_async_copy(
            table_hbm.at[idxall_v.at[pl.ds(0, 3 * cp)]],
            rows_v.at[buf], gsems[buf]).wait()

    def wb_start(c, buf):
        base = wbase + c * cp
        pltpu.make_async_copy(obuf_v.at[buf], out_hbm.at[pl.ds(base, cp)],
                              wsems[buf]).start()

    def wb_wait(buf):
        pltpu.make_async_copy(obuf_v.at[buf],
                              out_hbm.at[pl.ds(wbase, cp)],
                              wsems[buf]).wait()

    def compute(c, buf):
        def pbody(p, _):
            wv = wtsall_v[pl.ds(3 * cp * c + 3 * p, 16)]
            w0 = wv[0]
            w1 = wv[1]
            w2 = wv[2]
            r0 = 3 * p
            for j in range(f1 // 16):
                sl = pl.ds(j * 16, 16)
                acc = rows_v[buf, r0, sl] * w0
                acc += rows_v[buf, r0 + 1, sl] * w1
                acc += rows_v[buf, r0 + 2, sl] * w2
                obuf_v[buf, p, sl] = acc
            return ()

        lax.fori_loop(0, cp, pbody, (), unroll=4)

    fire(0, 0)

    def pair(t, _):
        for b in range(2):
            c = 2 * t + b

            @pl.when(c + 1 < nchunks)
            def _():
                fire(c + 1, 1 - b)

            gather_wait(b)

            @pl.when(c >= 2)
            def _():
                wb_wait(b)

            compute(c, b)
            wb_start(c, b)
        return ()

    lax.fori_loop(0, nchunks // 2, pair, (), unroll=False)
    wb_wait(0)
    wb_wait(1)


def _stage2_body(p1_ref, interp_ref, w1at_ref, b1_ref, y1_ref, s_ref, ss_ref):
    b = pl.program_id(0)
    n = pl.program_id(1)
    y1 = jnp.dot(p1_ref[0], w1at_ref[...],
                 preferred_element_type=jnp.float32, precision=_HI)
    y1 = y1 + interp_ref[0] + b1_ref[...]
    y1_ref[0] = y1

    @pl.when((b == 0) & (n == 0))
    def _():
        s_ref[...] = jnp.zeros_like(s_ref)
        ss_ref[...] = jnp.zeros_like(ss_ref)

    s_ref[...] += jnp.sum(y1, axis=0, keepdims=True)
    ss_ref[...] += jnp.sum(y1 * y1, axis=0, keepdims=True)


def _bn_affine(s, ss, g, beta, minv):
    mean = s * minv
    var = ss * minv - mean * mean
    inv = jax.lax.rsqrt(var + 1e-5)
    scale = g * inv
    shift = beta - mean * scale
    return scale, shift


def _stage3_body(y1_ref, s_ref, ss_ref, g1_ref, bt1_ref, w2t_ref, b2_ref,
                 y2_ref, s2_ref, ss2_ref, *, minv):
    b = pl.program_id(0)
    n = pl.program_id(1)
    scale, shift = _bn_affine(s_ref[...], ss_ref[...], g1_ref[...],
                              bt1_ref[...], minv)
    h = jnp.maximum(y1_ref[0] * scale + shift, 0.0)
    y2 = jnp.dot(h, w2t_ref[...], preferred_element_type=jnp.float32,
                 precision=_HI)
    y2 = y2 + b2_ref[...]
    y2_ref[0] = y2

    @pl.when((b == 0) & (n == 0))
    def _():
        s2_ref[...] = jnp.zeros_like(s2_ref)
        ss2_ref[...] = jnp.zeros_like(ss2_ref)

    s2_ref[...] += jnp.sum(y2, axis=0, keepdims=True)
    ss2_ref[...] += jnp.sum(y2 * y2, axis=0, keepdims=True)


def _stage4_body(y2_ref, s2_ref, ss2_ref, g2_ref, bt2_ref, out_ref, *, minv):
    scale, shift = _bn_affine(s2_ref[...], ss2_ref[...], g2_ref[...],
                              bt2_ref[...], minv)
    out_ref[0] = jnp.maximum(y2_ref[0] * scale + shift, 0.0)


def kernel(xyz1, xyz2, points1, points2, W1, b1, g1, beta1, W2, b2, g2, beta2):
    B, N, _ = xyz1.shape
    S = xyz2.shape[1]
    C1 = points1.shape[2]
    C2 = points2.shape[2]
    F1 = W1.shape[0]
    F2 = W2.shape[0]
    TN = 1024
    NT = N // TN
    minv = 1.0 / (B * N)

    x2t = jnp.transpose(xyz2, (0, 2, 1))
    w1at = jnp.transpose(W1[:, :C1])
    w1bt = jnp.transpose(W1[:, C1:])
    w2t = jnp.transpose(W2)
    b1r = b1.reshape(1, F1)
    g1r = g1.reshape(1, F1)
    bt1r = beta1.reshape(1, F1)
    b2r = b2.reshape(1, F2)
    g2r = g2.reshape(1, F2)
    bt2r = beta2.reshape(1, F2)
    f32 = jnp.float32

    idx3, wt3 = pl.pallas_call(
        functools.partial(_topk_body, S=S),
        grid=(B, NT),
        in_specs=[
            pl.BlockSpec((1, TN, 3), lambda b, n: (b, n, 0)),
            pl.BlockSpec((1, 3, S), lambda b, n: (b, 0, 0)),
        ],
        out_specs=[
            pl.BlockSpec((1, TN, 3), lambda b, n: (b, n, 0)),
            pl.BlockSpec((1, TN, 3), lambda b, n: (b, n, 0)),
        ],
        out_shape=[
            jax.ShapeDtypeStruct((B, N, 3), jnp.int32),
            jax.ShapeDtypeStruct((B, N, 3), f32),
        ],
    )(xyz1, x2t)

    p2w = pl.pallas_call(
        _p2w_body,
        grid=(B,),
        in_specs=[
            pl.BlockSpec((1, S, C2), lambda b: (b, 0, 0)),
            pl.BlockSpec((C2, F1), lambda b: (0, 0)),
        ],
        out_specs=pl.BlockSpec((1, S, F1), lambda b: (b, 0, 0)),
        out_shape=jax.ShapeDtypeStruct((B, S, F1), f32),
    )(points2, w1bt)

    table = p2w.reshape(B * S, F1)
    fidx = idx3.reshape(B * N * 3)
    fwts = wt3.reshape(B * N * 3)

    info = plsc.get_sparse_core_info()
    nc, ns = info.num_cores, info.num_subcores
    nw = nc * ns
    pts_per_w = (B * N) // nw
    cp = 16
    mesh = plsc.VectorSubcoreMesh(core_axis_name="c", subcore_axis_name="s")
    interp = pl.kernel(
        functools.partial(_sc_gather, pts_per_w=pts_per_w, cp=cp, f1=F1,
                          nc=nc),
        mesh=mesh,
        out_type=jax.ShapeDtypeStruct((B * N, F1), f32),
        scratch_types=[
            pltpu.VMEM((3 * pts_per_w,), jnp.int32),
            pltpu.VMEM((3 * pts_per_w + 16,), f32),
            pltpu.VMEM((2, 3 * cp, F1), f32),
            pltpu.VMEM((2, cp, F1), f32),
            pltpu.SemaphoreType.DMA,
            pltpu.SemaphoreType.DMA,
            pltpu.SemaphoreType.DMA,
            pltpu.SemaphoreType.DMA,
        ],
    )(table, fidx, fwts)

    interp3 = interp.reshape(B, N, F1)

    y1, s1, ss1 = pl.pallas_call(
        _stage2_body,
        grid=(B, NT),
        in_specs=[
            pl.BlockSpec((1, TN, C1), lambda b, n: (b, n, 0)),
            pl.BlockSpec((1, TN, F1), lambda b, n: (b, n, 0)),
            pl.BlockSpec((C1, F1), lambda b, n: (0, 0)),
            pl.BlockSpec((1, F1), lambda b, n: (0, 0)),
        ],
        out_specs=[
            pl.BlockSpec((1, TN, F1), lambda b, n: (b, n, 0)),
            pl.BlockSpec((1, F1), lambda b, n: (0, 0)),
            pl.BlockSpec((1, F1), lambda b, n: (0, 0)),
        ],
        out_shape=[
            jax.ShapeDtypeStruct((B, N, F1), f32),
            jax.ShapeDtypeStruct((1, F1), f32),
            jax.ShapeDtypeStruct((1, F1), f32),
        ],
    )(points1, interp3, w1at, b1r)

    y2, s2, ss2 = pl.pallas_call(
        functools.partial(_stage3_body, minv=minv),
        grid=(B, NT),
        in_specs=[
            pl.BlockSpec((1, TN, F1), lambda b, n: (b, n, 0)),
            pl.BlockSpec((1, F1), lambda b, n: (0, 0)),
            pl.BlockSpec((1, F1), lambda b, n: (0, 0)),
            pl.BlockSpec((1, F1), lambda b, n: (0, 0)),
            pl.BlockSpec((1, F1), lambda b, n: (0, 0)),
            pl.BlockSpec((F1, F2), lambda b, n: (0, 0)),
            pl.BlockSpec((1, F2), lambda b, n: (0, 0)),
        ],
        out_specs=[
            pl.BlockSpec((1, TN, F2), lambda b, n: (b, n, 0)),
            pl.BlockSpec((1, F2), lambda b, n: (0, 0)),
            pl.BlockSpec((1, F2), lambda b, n: (0, 0)),
        ],
        out_shape=[
            jax.ShapeDtypeStruct((B, N, F2), f32),
            jax.ShapeDtypeStruct((1, F2), f32),
            jax.ShapeDtypeStruct((1, F2), f32),
        ],
    )(y1, s1, ss1, g1r, bt1r, w2t, b2r)

    out = pl.pallas_call(
        functools.partial(_stage4_body, minv=minv),
        grid=(B, NT),
        in_specs=[
            pl.BlockSpec((1, TN, F2), lambda b, n: (b, n, 0)),
            pl.BlockSpec((1, F2), lambda b, n: (0, 0)),
            pl.BlockSpec((1, F2), lambda b, n: (0, 0)),
            pl.BlockSpec((1, F2), lambda b, n: (0, 0)),
            pl.BlockSpec((1, F2), lambda b, n: (0, 0)),
        ],
        out_specs=pl.BlockSpec((1, TN, F2), lambda b, n: (b, n, 0)),
        out_shape=jax.ShapeDtypeStruct((B, N, F2), f32),
    )(y2, s2, ss2, g2r, bt2r)

    return out


# speedup vs baseline: 1.0253x; 1.0253x over previous
import functools

import jax
import jax.numpy as jnp
from jax import lax
from jax.experimental import pallas as pl
from jax.experimental.pallas import tpu as pltpu
from jax.experimental.pallas import tpu_sc as plsc

_HI = jax.lax.Precision.HIGHEST


def _topk_body(x1_ref, x2t_ref, idx_ref, wt_ref, *, S):
    b = pl.program_id(0)
    x1 = x1_ref[0]
    x2t = x2t_ref[0]
    d0 = x1[:, 0:1] - x2t[0:1, :]
    d1 = x1[:, 1:2] - x2t[1:2, :]
    d2c = x1[:, 2:3] - x2t[2:3, :]
    d = d0 * d0 + d1 * d1 + d2c * d2c

    inf = jnp.float32(jnp.inf)
    iota = lax.broadcasted_iota(jnp.int32, d.shape, 1)
    m1 = jnp.min(d, axis=1, keepdims=True)
    i1 = jnp.min(jnp.where(d <= m1, iota, S), axis=1, keepdims=True)
    dm2 = jnp.where(d <= m1, inf, d)
    m2 = jnp.min(dm2, axis=1, keepdims=True)
    i2 = jnp.min(jnp.where(dm2 <= m2, iota, S), axis=1, keepdims=True)
    dm3 = jnp.where(d <= m2, inf, d)
    m3 = jnp.min(dm3, axis=1, keepdims=True)
    i3 = jnp.min(jnp.where(dm3 <= m3, iota, S), axis=1, keepdims=True)

    r1 = 1.0 / (m1 + 1e-8)
    r2 = 1.0 / (m2 + 1e-8)
    r3 = 1.0 / (m3 + 1e-8)
    norm = r1 + r2 + r3
    base = b * S
    idx_ref[0] = jnp.concatenate([i1 + base, i2 + base, i3 + base], axis=1)
    wt_ref[0] = jnp.concatenate([r1, r2, r3], axis=1) / norm


def _p2w_body(p2_ref, w1bt_ref, out_ref):
    out_ref[0] = jnp.dot(p2_ref[0], w1bt_ref[...],
                         preferred_element_type=jnp.float32, precision=_HI)


def _sc_gather(table_hbm, idx_hbm, wts_hbm, out_hbm,
               idxall_v, wtsall_v, rows_v, obuf_v, gs0, gs1, ws0, ws1, *,
               pts_per_w, cp, f1, nc):
    wid = lax.axis_index("s") * nc + lax.axis_index("c")
    wbase = wid * pts_per_w
    nchunks = pts_per_w // cp
    gsems = (gs0, gs1)
    wsems = (ws0, ws1)

    pltpu.sync_copy(idx_hbm.at[pl.ds(3 * wbase, 3 * pts_per_w)], idxall_v)
    pltpu.sync_copy(wts_hbm.at[pl.ds(3 * wbase, 3 * pts_per_w)],
                    wtsall_v.at[pl.ds(0, 3 * pts_per_w)])

    def fire(c, buf):
        pltpu.make_async_copy(
            table_hbm.at[idxall_v.at[pl.ds(3 * cp * c, 3 * cp)]],
            rows_v.at[buf], gsems[buf]).start()

    def gather_wait(buf):
        pltpu.make_async_copy(
            table_hbm.at[idxall_v.at[pl.ds(0, 3 * cp)]],
            rows_v.at[buf], gsems[buf]).wait()

    def wb_start(c, buf):
        base = wbase + c * cp
        pltpu.make_async_copy(obuf_v.at[buf], out_hbm.at[pl.ds(base, cp)],
                              wsems[buf]).start()

    def wb_wait(buf):
        pltpu.make_async_copy(obuf_v.at[buf],
                              out_hbm.at[pl.ds(wbase, cp)],
                              wsems[buf]).wait()

    def compute(c, buf):
        def pbody(p, _):
            wv = wtsall_v[pl.ds(3 * cp * c + 3 * p, 16)]
            w0 = wv[0]
            w1 = wv[1]
            w2 = wv[2]
            r0 = 3 * p
            for j in range(f1 // 16):
                sl = pl.ds(j * 16, 16)
                acc = rows_v[buf, r0, sl] * w0
                acc += rows_v[buf, r0 + 1, sl] * w1
                acc += rows_v[buf, r0 + 2, sl] * w2
                obuf_v[buf, p, sl] = acc
            return ()

        lax.fori_loop(0, cp, pbody, (), unroll=False)

    fire(0, 0)

    def pair(t, _):
        for b in range(2):
            c = 2 * t + b

            @pl.when(c + 1 < nchunks)
            def _():
                fire(c + 1, 1 - b)

            gather_wait(b)

            @pl.when(c >= 2)
            def _():
                wb_wait(b)

            compute(c, b)
            wb_start(c, b)
        return ()

    lax.fori_loop(0, nchunks // 2, pair, (), unroll=False)
    wb_wait(0)
    wb_wait(1)


def _stage2_body(p1_ref, interp_ref, w1at_ref, b1_ref, y1_ref, s_ref, ss_ref):
    b = pl.program_id(0)
    n = pl.program_id(1)
    y1 = jnp.dot(p1_ref[0], w1at_ref[...],
                 preferred_element_type=jnp.float32, precision=_HI)
    y1 = y1 + interp_ref[0] + b1_ref[...]
    y1_ref[0] = y1

    @pl.when((b == 0) & (n == 0))
    def _():
        s_ref[...] = jnp.zeros_like(s_ref)
        ss_ref[...] = jnp.zeros_like(ss_ref)

    s_ref[...] += jnp.sum(y1, axis=0, keepdims=True)
    ss_ref[...] += jnp.sum(y1 * y1, axis=0, keepdims=True)


def _bn_affine(s, ss, g, beta, minv):
    mean = s * minv
    var = ss * minv - mean * mean
    inv = jax.lax.rsqrt(var + 1e-5)
    scale = g * inv
    shift = beta - mean * scale
    return scale, shift


def _stage3_body(y1_ref, s_ref, ss_ref, g1_ref, bt1_ref, w2t_ref, b2_ref,
                 y2_ref, s2_ref, ss2_ref, *, minv):
    b = pl.program_id(0)
    n = pl.program_id(1)
    scale, shift = _bn_affine(s_ref[...], ss_ref[...], g1_ref[...],
                              bt1_ref[...], minv)
    h = jnp.maximum(y1_ref[0] * scale + shift, 0.0)
    y2 = jnp.dot(h, w2t_ref[...], preferred_element_type=jnp.float32,
                 precision=_HI)
    y2 = y2 + b2_ref[...]
    y2_ref[0] = y2

    @pl.when((b == 0) & (n == 0))
    def _():
        s2_ref[...] = jnp.zeros_like(s2_ref)
        ss2_ref[...] = jnp.zeros_like(ss2_ref)

    s2_ref[...] += jnp.sum(y2, axis=0, keepdims=True)
    ss2_ref[...] += jnp.sum(y2 * y2, axis=0, keepdims=True)


def _stage4_body(y2_ref, s2_ref, ss2_ref, g2_ref, bt2_ref, out_ref, *, minv):
    scale, shift = _bn_affine(s2_ref[...], ss2_ref[...], g2_ref[...],
                              bt2_ref[...], minv)
    out_ref[0] = jnp.maximum(y2_ref[0] * scale + shift, 0.0)


def kernel(xyz1, xyz2, points1, points2, W1, b1, g1, beta1, W2, b2, g2, beta2):
    B, N, _ = xyz1.shape
    S = xyz2.shape[1]
    C1 = points1.shape[2]
    C2 = points2.shape[2]
    F1 = W1.shape[0]
    F2 = W2.shape[0]
    TN = 1024
    NT = N // TN
    minv = 1.0 / (B * N)

    x2t = jnp.transpose(xyz2, (0, 2, 1))
    w1at = jnp.transpose(W1[:, :C1])
    w1bt = jnp.transpose(W1[:, C1:])
    w2t = jnp.transpose(W2)
    b1r = b1.reshape(1, F1)
    g1r = g1.reshape(1, F1)
    bt1r = beta1.reshape(1, F1)
    b2r = b2.reshape(1, F2)
    g2r = g2.reshape(1, F2)
    bt2r = beta2.reshape(1, F2)
    f32 = jnp.float32

    idx3, wt3 = pl.pallas_call(
        functools.partial(_topk_body, S=S),
        grid=(B, NT),
        in_specs=[
            pl.BlockSpec((1, TN, 3), lambda b, n: (b, n, 0)),
            pl.BlockSpec((1, 3, S), lambda b, n: (b, 0, 0)),
        ],
        out_specs=[
            pl.BlockSpec((1, TN, 3), lambda b, n: (b, n, 0)),
            pl.BlockSpec((1, TN, 3), lambda b, n: (b, n, 0)),
        ],
        out_shape=[
            jax.ShapeDtypeStruct((B, N, 3), jnp.int32),
            jax.ShapeDtypeStruct((B, N, 3), f32),
        ],
    )(xyz1, x2t)

    p2w = pl.pallas_call(
        _p2w_body,
        grid=(B,),
        in_specs=[
            pl.BlockSpec((1, S, C2), lambda b: (b, 0, 0)),
            pl.BlockSpec((C2, F1), lambda b: (0, 0)),
        ],
        out_specs=pl.BlockSpec((1, S, F1), lambda b: (b, 0, 0)),
        out_shape=jax.ShapeDtypeStruct((B, S, F1), f32),
    )(points2, w1bt)

    table = p2w.reshape(B * S, F1)
    fidx = idx3.reshape(B * N * 3)
    fwts = wt3.reshape(B * N * 3)

    info = plsc.get_sparse_core_info()
    nc, ns = info.num_cores, info.num_subcores
    nw = nc * ns
    pts_per_w = (B * N) // nw
    cp = 16
    mesh = plsc.VectorSubcoreMesh(core_axis_name="c", subcore_axis_name="s")
    interp = pl.kernel(
        functools.partial(_sc_gather, pts_per_w=pts_per_w, cp=cp, f1=F1,
                          nc=nc),
        mesh=mesh,
        out_type=jax.ShapeDtypeStruct((B * N, F1), f32),
        scratch_types=[
            pltpu.VMEM((3 * pts_per_w,), jnp.int32),
            pltpu.VMEM((3 * pts_per_w + 16,), f32),
            pltpu.VMEM((2, 3 * cp, F1), f32),
            pltpu.VMEM((2, cp, F1), f32),
            pltpu.SemaphoreType.DMA,
            pltpu.SemaphoreType.DMA,
            pltpu.SemaphoreType.DMA,
            pltpu.SemaphoreType.DMA,
        ],
    )(table, fidx, fwts)

    interp3 = interp.reshape(B, N, F1)

    y1, s1, ss1 = pl.pallas_call(
        _stage2_body,
        grid=(B, NT),
        in_specs=[
            pl.BlockSpec((1, TN, C1), lambda b, n: (b, n, 0)),
            pl.BlockSpec((1, TN, F1), lambda b, n: (b, n, 0)),
            pl.BlockSpec((C1, F1), lambda b, n: (0, 0)),
            pl.BlockSpec((1, F1), lambda b, n: (0, 0)),
        ],
        out_specs=[
            pl.BlockSpec((1, TN, F1), lambda b, n: (b, n, 0)),
            pl.BlockSpec((1, F1), lambda b, n: (0, 0)),
            pl.BlockSpec((1, F1), lambda b, n: (0, 0)),
        ],
        out_shape=[
            jax.ShapeDtypeStruct((B, N, F1), f32),
            jax.ShapeDtypeStruct((1, F1), f32),
            jax.ShapeDtypeStruct((1, F1), f32),
        ],
    )(points1, interp3, w1at, b1r)

    y2, s2, ss2 = pl.pallas_call(
        functools.partial(_stage3_body, minv=minv),
        grid=(B, NT),
        in_specs=[
            pl.BlockSpec((1, TN, F1), lambda b, n: (b, n, 0)),
            pl.BlockSpec((1, F1), lambda b, n: (0, 0)),
            pl.BlockSpec((1, F1), lambda b, n: (0, 0)),
            pl.BlockSpec((1, F1), lambda b, n: (0, 0)),
            pl.BlockSpec((1, F1), lambda b, n: (0, 0)),
            pl.BlockSpec((F1, F2), lambda b, n: (0, 0)),
            pl.BlockSpec((1, F2), lambda b, n: (0, 0)),
        ],
        out_specs=[
            pl.BlockSpec((1, TN, F2), lambda b, n: (b, n, 0)),
            pl.BlockSpec((1, F2), lambda b, n: (0, 0)),
            pl.BlockSpec((1, F2), lambda b, n: (0, 0)),
        ],
        out_shape=[
            jax.ShapeDtypeStruct((B, N, F2), f32),
            jax.ShapeDtypeStruct((1, F2), f32),
            jax.ShapeDtypeStruct((1, F2), f32),
        ],
    )(y1, s1, ss1, g1r, bt1r, w2t, b2r)

    out = pl.pallas_call(
        functools.partial(_stage4_body, minv=minv),
        grid=(B, NT),
        in_specs=[
            pl.BlockSpec((1, TN, F2), lambda b, n: (b, n, 0)),
            pl.BlockSpec((1, F2), lambda b, n: (0, 0)),
            pl.BlockSpec((1, F2), lambda b, n: (0, 0)),
            pl.BlockSpec((1, F2), lambda b, n: (0, 0)),
            pl.BlockSpec((1, F2), lambda b, n: (0, 0)),
        ],
        out_specs=pl.BlockSpec((1, TN, F2), lambda b, n: (b, n, 0)),
        out_shape=jax.ShapeDtypeStruct((B, N, F2), f32),
    )(y2, s2, ss2, g2r, bt2r)

    return out


# speedup vs baseline: 1.0443x; 1.0186x over previous
import functools

import jax
import jax.numpy as jnp
from jax import lax
from jax.experimental import pallas as pl
from jax.experimental.pallas import tpu as pltpu
from jax.experimental.pallas import tpu_sc as plsc

_HI = jax.lax.Precision.HIGHEST


def _topk_body(x1_ref, x2t_ref, idx_ref, wt_ref, *, S):
    b = pl.program_id(0)
    x1 = x1_ref[0]
    x2t = x2t_ref[0]
    d0 = x1[:, 0:1] - x2t[0:1, :]
    d1 = x1[:, 1:2] - x2t[1:2, :]
    d2c = x1[:, 2:3] - x2t[2:3, :]
    d = d0 * d0 + d1 * d1 + d2c * d2c

    inf = jnp.float32(jnp.inf)
    iota = lax.broadcasted_iota(jnp.int32, d.shape, 1)
    m1 = jnp.min(d, axis=1, keepdims=True)
    i1 = jnp.min(jnp.where(d <= m1, iota, S), axis=1, keepdims=True)
    dm2 = jnp.where(d <= m1, inf, d)
    m2 = jnp.min(dm2, axis=1, keepdims=True)
    i2 = jnp.min(jnp.where(dm2 <= m2, iota, S), axis=1, keepdims=True)
    dm3 = jnp.where(d <= m2, inf, d)
    m3 = jnp.min(dm3, axis=1, keepdims=True)
    i3 = jnp.min(jnp.where(dm3 <= m3, iota, S), axis=1, keepdims=True)

    r1 = 1.0 / (m1 + 1e-8)
    r2 = 1.0 / (m2 + 1e-8)
    r3 = 1.0 / (m3 + 1e-8)
    norm = r1 + r2 + r3
    base = b * S
    idx_ref[0] = jnp.concatenate([i1 + base, i2 + base, i3 + base], axis=1)
    wt_ref[0] = jnp.concatenate([r1, r2, r3], axis=1) / norm


def _p2w_body(p2_ref, w1bt_ref, out_ref):
    out_ref[0] = jnp.dot(p2_ref[0], w1bt_ref[...],
                         preferred_element_type=jnp.float32, precision=_HI)


def _sc_gather(table_hbm, idx_hbm, wts_hbm, out_hbm,
               idxall_v, wtsall_v, rows_v, obuf_v, gs0, gs1, ws0, ws1, *,
               pts_per_w, cp, f1, nc):
    wid = lax.axis_index("s") * nc + lax.axis_index("c")
    wbase = wid * pts_per_w
    nchunks = pts_per_w // cp
    gsems = (gs0, gs1)
    wsems = (ws0, ws1)

    pltpu.sync_copy(idx_hbm.at[pl.ds(3 * wbase, 3 * pts_per_w)], idxall_v)
    pltpu.sync_copy(wts_hbm.at[pl.ds(3 * wbase, 3 * pts_per_w)],
                    wtsall_v.at[pl.ds(0, 3 * pts_per_w)])

    def fire(c, buf):
        pltpu.make_async_copy(
            table_hbm.at[idxall_v.at[pl.ds(3 * cp * c, 3 * cp)]],
            rows_v.at[buf], gsems[buf]).start()

    def gather_wait(buf):
        pltpu.make_async_copy(
            table_hbm.at[idxall_v.at[pl.ds(0, 3 * cp)]],
            rows_v.at[buf], gsems[buf]).wait()

    def wb_start(c, buf):
        base = wbase + c * cp
        pltpu.make_async_copy(obuf_v.at[buf], out_hbm.at[pl.ds(base, cp)],
                              wsems[buf]).start()

    def wb_wait(buf):
        pltpu.make_async_copy(obuf_v.at[buf],
                              out_hbm.at[pl.ds(wbase, cp)],
                              wsems[buf]).wait()

    def compute(c, buf):
        def pbody(p, _):
            wv = wtsall_v[pl.ds(3 * cp * c + 3 * p, 16)]
            w0 = wv[0]
            w1 = wv[1]
            w2 = wv[2]
            r0 = 3 * p
            for j in range(f1 // 16):
                sl = pl.ds(j * 16, 16)
                acc = rows_v[buf, r0, sl] * w0
                acc += rows_v[buf, r0 + 1, sl] * w1
                acc += rows_v[buf, r0 + 2, sl] * w2
                obuf_v[buf, p, sl] = acc
            return ()

        lax.fori_loop(0, cp, pbody, (), unroll=False)

    fire(0, 0)

    def pair(t, _):
        for b in range(2):
            c = 2 * t + b

            @pl.when(c + 1 < nchunks)
            def _():
                fire(c + 1, 1 - b)

            gather_wait(b)

            @pl.when(c >= 2)
            def _():
                wb_wait(b)

            compute(c, b)
            wb_start(c, b)
        return ()

    lax.fori_loop(0, nchunks // 2, pair, (), unroll=False)
    wb_wait(0)
    wb_wait(1)


def _stage2a_body(p1_ref, w1at_ref, b1_ref, y1p_ref):
    y1p = jnp.dot(p1_ref[0], w1at_ref[...],
                  preferred_element_type=jnp.float32, precision=_HI)
    y1p_ref[0] = y1p + b1_ref[...]


def _stage2_body(y1p_ref, interp_ref, y1_ref, s_ref, ss_ref):
    b = pl.program_id(0)
    n = pl.program_id(1)
    y1 = y1p_ref[0] + interp_ref[0]
    y1_ref[0] = y1

    @pl.when((b == 0) & (n == 0))
    def _():
        s_ref[...] = jnp.zeros_like(s_ref)
        ss_ref[...] = jnp.zeros_like(ss_ref)

    s_ref[...] += jnp.sum(y1, axis=0, keepdims=True)
    ss_ref[...] += jnp.sum(y1 * y1, axis=0, keepdims=True)


def _bn_affine(s, ss, g, beta, minv):
    mean = s * minv
    var = ss * minv - mean * mean
    inv = jax.lax.rsqrt(var + 1e-5)
    scale = g * inv
    shift = beta - mean * scale
    return scale, shift


def _stage3_body(y1_ref, s_ref, ss_ref, g1_ref, bt1_ref, w2t_ref, b2_ref,
                 y2_ref, s2_ref, ss2_ref, *, minv):
    b = pl.program_id(0)
    n = pl.program_id(1)
    scale, shift = _bn_affine(s_ref[...], ss_ref[...], g1_ref[...],
                              bt1_ref[...], minv)
    h = jnp.maximum(y1_ref[0] * scale + shift, 0.0)
    y2 = jnp.dot(h, w2t_ref[...], preferred_element_type=jnp.float32,
                 precision=_HI)
    y2 = y2 + b2_ref[...]
    y2_ref[0] = y2

    @pl.when((b == 0) & (n == 0))
    def _():
        s2_ref[...] = jnp.zeros_like(s2_ref)
        ss2_ref[...] = jnp.zeros_like(ss2_ref)

    s2_ref[...] += jnp.sum(y2, axis=0, keepdims=True)
    ss2_ref[...] += jnp.sum(y2 * y2, axis=0, keepdims=True)


def _stage4_body(y2_ref, s2_ref, ss2_ref, g2_ref, bt2_ref, out_ref, *, minv):
    scale, shift = _bn_affine(s2_ref[...], ss2_ref[...], g2_ref[...],
                              bt2_ref[...], minv)
    out_ref[0] = jnp.maximum(y2_ref[0] * scale + shift, 0.0)


def kernel(xyz1, xyz2, points1, points2, W1, b1, g1, beta1, W2, b2, g2, beta2):
    B, N, _ = xyz1.shape
    S = xyz2.shape[1]
    C1 = points1.shape[2]
    C2 = points2.shape[2]
    F1 = W1.shape[0]
    F2 = W2.shape[0]
    TN = 1024
    NT = N // TN
    minv = 1.0 / (B * N)

    x2t = jnp.transpose(xyz2, (0, 2, 1))
    w1at = jnp.transpose(W1[:, :C1])
    w1bt = jnp.transpose(W1[:, C1:])
    w2t = jnp.transpose(W2)
    b1r = b1.reshape(1, F1)
    g1r = g1.reshape(1, F1)
    bt1r = beta1.reshape(1, F1)
    b2r = b2.reshape(1, F2)
    g2r = g2.reshape(1, F2)
    bt2r = beta2.reshape(1, F2)
    f32 = jnp.float32

    idx3, wt3 = pl.pallas_call(
        functools.partial(_topk_body, S=S),
        grid=(B, NT),
        in_specs=[
            pl.BlockSpec((1, TN, 3), lambda b, n: (b, n, 0)),
            pl.BlockSpec((1, 3, S), lambda b, n: (b, 0, 0)),
        ],
        out_specs=[
            pl.BlockSpec((1, TN, 3), lambda b, n: (b, n, 0)),
            pl.BlockSpec((1, TN, 3), lambda b, n: (b, n, 0)),
        ],
        out_shape=[
            jax.ShapeDtypeStruct((B, N, 3), jnp.int32),
            jax.ShapeDtypeStruct((B, N, 3), f32),
        ],
    )(xyz1, x2t)

    p2w = pl.pallas_call(
        _p2w_body,
        grid=(B,),
        in_specs=[
            pl.BlockSpec((1, S, C2), lambda b: (b, 0, 0)),
            pl.BlockSpec((C2, F1), lambda b: (0, 0)),
        ],
        out_specs=pl.BlockSpec((1, S, F1), lambda b: (b, 0, 0)),
        out_shape=jax.ShapeDtypeStruct((B, S, F1), f32),
    )(points2, w1bt)

    table = p2w.reshape(B * S, F1)
    fidx = idx3.reshape(B * N * 3)
    fwts = wt3.reshape(B * N * 3)

    info = plsc.get_sparse_core_info()
    nc, ns = info.num_cores, info.num_subcores
    nw = nc * ns
    pts_per_w = (B * N) // nw
    cp = 16
    mesh = plsc.VectorSubcoreMesh(core_axis_name="c", subcore_axis_name="s")
    interp = pl.kernel(
        functools.partial(_sc_gather, pts_per_w=pts_per_w, cp=cp, f1=F1,
                          nc=nc),
        mesh=mesh,
        out_type=jax.ShapeDtypeStruct((B * N, F1), f32),
        scratch_types=[
            pltpu.VMEM((3 * pts_per_w,), jnp.int32),
            pltpu.VMEM((3 * pts_per_w + 16,), f32),
            pltpu.VMEM((2, 3 * cp, F1), f32),
            pltpu.VMEM((2, cp, F1), f32),
            pltpu.SemaphoreType.DMA,
            pltpu.SemaphoreType.DMA,
            pltpu.SemaphoreType.DMA,
            pltpu.SemaphoreType.DMA,
        ],
    )(table, fidx, fwts)

    interp3 = interp.reshape(B, N, F1)

    y1p = pl.pallas_call(
        _stage2a_body,
        grid=(B, NT),
        in_specs=[
            pl.BlockSpec((1, TN, C1), lambda b, n: (b, n, 0)),
            pl.BlockSpec((C1, F1), lambda b, n: (0, 0)),
            pl.BlockSpec((1, F1), lambda b, n: (0, 0)),
        ],
        out_specs=pl.BlockSpec((1, TN, F1), lambda b, n: (b, n, 0)),
        out_shape=jax.ShapeDtypeStruct((B, N, F1), f32),
    )(points1, w1at, b1r)

    y1, s1, ss1 = pl.pallas_call(
        _stage2_body,
        grid=(B, NT),
        in_specs=[
            pl.BlockSpec((1, TN, F1), lambda b, n: (b, n, 0)),
            pl.BlockSpec((1, TN, F1), lambda b, n: (b, n, 0)),
        ],
        out_specs=[
            pl.BlockSpec((1, TN, F1), lambda b, n: (b, n, 0)),
            pl.BlockSpec((1, F1), lambda b, n: (0, 0)),
            pl.BlockSpec((1, F1), lambda b, n: (0, 0)),
        ],
        out_shape=[
            jax.ShapeDtypeStruct((B, N, F1), f32),
            jax.ShapeDtypeStruct((1, F1), f32),
            jax.ShapeDtypeStruct((1, F1), f32),
        ],
    )(y1p, interp3)

    y2, s2, ss2 = pl.pallas_call(
        functools.partial(_stage3_body, minv=minv),
        grid=(B, NT),
        in_specs=[
            pl.BlockSpec((1, TN, F1), lambda b, n: (b, n, 0)),
            pl.BlockSpec((1, F1), lambda b, n: (0, 0)),
            pl.BlockSpec((1, F1), lambda b, n: (0, 0)),
            pl.BlockSpec((1, F1), lambda b, n: (0, 0)),
            pl.BlockSpec((1, F1), lambda b, n: (0, 0)),
            pl.BlockSpec((F1, F2), lambda b, n: (0, 0)),
            pl.BlockSpec((1, F2), lambda b, n: (0, 0)),
        ],
        out_specs=[
            pl.BlockSpec((1, TN, F2), lambda b, n: (b, n, 0)),
            pl.BlockSpec((1, F2), lambda b, n: (0, 0)),
            pl.BlockSpec((1, F2), lambda b, n: (0, 0)),
        ],
        out_shape=[
            jax.ShapeDtypeStruct((B, N, F2), f32),
            jax.ShapeDtypeStruct((1, F2), f32),
            jax.ShapeDtypeStruct((1, F2), f32),
        ],
    )(y1, s1, ss1, g1r, bt1r, w2t, b2r)

    out = pl.pallas_call(
        functools.partial(_stage4_body, minv=minv),
        grid=(B, NT),
        in_specs=[
            pl.BlockSpec((1, TN, F2), lambda b, n: (b, n, 0)),
            pl.BlockSpec((1, F2), lambda b, n: (0, 0)),
            pl.BlockSpec((1, F2), lambda b, n: (0, 0)),
            pl.BlockSpec((1, F2), lambda b, n: (0, 0)),
            pl.BlockSpec((1, F2), lambda b, n: (0, 0)),
        ],
        out_specs=pl.BlockSpec((1, TN, F2), lambda b, n: (b, n, 0)),
        out_shape=jax.ShapeDtypeStruct((B, N, F2), f32),
    )(y2, s2, ss2, g2r, bt2r)

    return out
